# initial kernel scaffold (unmeasured)
import jax
import jax.numpy as jnp
from jax import lax
from jax.experimental import pallas as pl
from jax.experimental.pallas import tpu as pltpu

N_DEV = 4
M_PER = 2048
K = 8192
N_PER = 1024
K_BLK = 2048
N_K = K // K_BLK

OFF = (2, 1, 3, 0)


def kernel(x, w_mat):
    x16 = x.astype(jnp.bfloat16)
    w16 = w_mat.astype(jnp.bfloat16)

    p = lax.axis_index("i").astype(jnp.int32)
    perm = (p + jnp.array(OFF, dtype=jnp.int32)) % N_DEV

    def body(perm_ref, x_ref, w_ref, out_ref, y_acc, send_buf, recv_buf,
             send_sems, recv_sems):
        j = pl.program_id(0)
        k = pl.program_id(1)
        me = lax.axis_index("i").astype(jnp.int32)

        @pl.when(jnp.logical_and(j == 0, k == 0))
        def _():
            barrier = pltpu.get_barrier_semaphore()
            for d in range(1, N_DEV):
                pl.semaphore_signal(
                    barrier, inc=1,
                    device_id=((me + d) % N_DEV,),
                    device_id_type=pl.DeviceIdType.MESH,
                )
            pl.semaphore_wait(barrier, N_DEV - 1)

        @pl.when(k == 0)
        def _():
            y_acc[...] = jnp.zeros_like(y_acc)
        y_acc[...] += jnp.dot(
            x_ref[...], w_ref[...], preferred_element_type=jnp.float32
        )

        @pl.when(k == N_K - 1)
        def _():
            c = perm_ref[j]

            @pl.when(c == me)
            def _():
                out_ref[pl.ds(me * M_PER, M_PER), :] = (
                    y_acc[...].astype(jnp.bfloat16)
                )

            @pl.when(c != me)
            def _():
                send_buf[j] = y_acc[...].astype(jnp.bfloat16)
                rdma = pltpu.make_async_remote_copy(
                    src_ref=send_buf.at[j],
                    dst_ref=recv_buf.at[me],
                    send_sem=send_sems.at[j],
                    recv_sem=recv_sems.at[me],
                    device_id=(c,),
                    device_id_type=pl.DeviceIdType.MESH,
                )
                rdma.start()

        @pl.when(jnp.logical_and(j == N_DEV - 1, k == N_K - 1))
        def _():
            for d in (2, 3, 1):
                o = (me + d) % N_DEV
                rdma = pltpu.make_async_remote_copy(
                    src_ref=send_buf.at[0],
                    dst_ref=recv_buf.at[o],
                    send_sem=send_sems.at[0],
                    recv_sem=recv_sems.at[o],
                    device_id=(me,),
                    device_id_type=pl.DeviceIdType.MESH,
                )
                rdma.wait_recv()
                out_ref[pl.ds(o * M_PER, M_PER), :] = recv_buf[o]
            for s in range(N_DEV - 1):
                rdma = pltpu.make_async_remote_copy(
                    src_ref=send_buf.at[s],
                    dst_ref=recv_buf.at[me],
                    send_sem=send_sems.at[s],
                    recv_sem=recv_sems.at[me],
                    device_id=(me,),
                    device_id_type=pl.DeviceIdType.MESH,
                )
                rdma.wait_send()

    grid_spec = pltpu.PrefetchScalarGridSpec(
        num_scalar_prefetch=1,
        grid=(N_DEV, N_K),
        in_specs=[
            pl.BlockSpec((M_PER, K_BLK), lambda j, k, perm_ref: (0, k)),
            pl.BlockSpec((K_BLK, N_PER), lambda j, k, perm_ref: (k, perm_ref[j])),
        ],
        out_specs=pl.BlockSpec(
            (N_DEV * M_PER, N_PER), lambda j, k, perm_ref: (0, 0)
        ),
        scratch_shapes=[
            pltpu.VMEM((M_PER, N_PER), jnp.float32),
            pltpu.VMEM((N_DEV - 1, M_PER, N_PER), jnp.bfloat16),
            pltpu.VMEM((N_DEV, M_PER, N_PER), jnp.bfloat16),
            pltpu.SemaphoreType.DMA((N_DEV - 1,)),
            pltpu.SemaphoreType.DMA((N_DEV,)),
        ],
    )

    out16 = pl.pallas_call(
        body,
        grid_spec=grid_spec,
        out_shape=jax.ShapeDtypeStruct((N_DEV * M_PER, N_PER), jnp.bfloat16),
        compiler_params=pltpu.CompilerParams(collective_id=0),
    )(perm, x16, w16)
    return out16.astype(jnp.float32)


# baseline (device time: 295568 ns/iter reference)
import jax
import jax.numpy as jnp
from jax import lax
from jax.experimental import pallas as pl
from jax.experimental.pallas import tpu as pltpu

N_DEV = 4
M_PER = 2048
K = 8192
N_PER = 1024
K_BLK = 1024
N_K = K // K_BLK

OFF = (2, 1, 3, 0)


def kernel(x, w_mat):
    x16 = x.astype(jnp.bfloat16)
    w16 = w_mat.astype(jnp.bfloat16)

    p = lax.axis_index("i").astype(jnp.int32)
    perm = (p + jnp.array(OFF, dtype=jnp.int32)) % N_DEV

    def body(perm_ref, x_ref, w_ref, out_ref, y_acc, send_buf,
             send_sems, recv_sems):
        j = pl.program_id(0)
        k = pl.program_id(1)
        me = lax.axis_index("i").astype(jnp.int32)

        @pl.when(jnp.logical_and(j == 0, k == 0))
        def _():
            barrier = pltpu.get_barrier_semaphore()
            for d in range(1, N_DEV):
                pl.semaphore_signal(
                    barrier, inc=1,
                    device_id=((me + d) % N_DEV,),
                    device_id_type=pl.DeviceIdType.MESH,
                )
            pl.semaphore_wait(barrier, N_DEV - 1)

        @pl.when(k == 0)
        def _():
            y_acc[...] = jnp.zeros_like(y_acc)
        y_acc[...] += jnp.dot(
            x_ref[...], w_ref[...], preferred_element_type=jnp.float32
        )

        @pl.when(k == N_K - 1)
        def _():
            c = perm_ref[j]

            @pl.when(c == me)
            def _():
                out_ref[pl.ds(me * M_PER, M_PER), :] = (
                    y_acc[...].astype(jnp.bfloat16)
                )

            @pl.when(c != me)
            def _():
                send_buf[j] = y_acc[...].astype(jnp.bfloat16)
                slot = (me - c - 1) % N_DEV
                rdma = pltpu.make_async_remote_copy(
                    src_ref=send_buf.at[j],
                    dst_ref=out_ref.at[pl.ds(me * M_PER, M_PER), :],
                    send_sem=send_sems.at[j],
                    recv_sem=recv_sems.at[slot],
                    device_id=(c,),
                    device_id_type=pl.DeviceIdType.MESH,
                )
                rdma.start()

        @pl.when(jnp.logical_and(j == N_DEV - 1, k == N_K - 1))
        def _():
            for d in (2, 3, 1):
                o = (me + d) % N_DEV
                slot = (o - me - 1) % N_DEV
                rdma = pltpu.make_async_remote_copy(
                    src_ref=send_buf.at[0],
                    dst_ref=out_ref.at[pl.ds(o * M_PER, M_PER), :],
                    send_sem=send_sems.at[0],
                    recv_sem=recv_sems.at[slot],
                    device_id=(me,),
                    device_id_type=pl.DeviceIdType.MESH,
                )
                rdma.wait_recv()
            for s in range(N_DEV - 1):
                rdma = pltpu.make_async_remote_copy(
                    src_ref=send_buf.at[s],
                    dst_ref=out_ref.at[pl.ds(me * M_PER, M_PER), :],
                    send_sem=send_sems.at[s],
                    recv_sem=recv_sems.at[0],
                    device_id=(me,),
                    device_id_type=pl.DeviceIdType.MESH,
                )
                rdma.wait_send()

    grid_spec = pltpu.PrefetchScalarGridSpec(
        num_scalar_prefetch=1,
        grid=(N_DEV, N_K),
        in_specs=[
            pl.BlockSpec((M_PER, K_BLK), lambda j, k, perm_ref: (0, k)),
            pl.BlockSpec((K_BLK, N_PER), lambda j, k, perm_ref: (k, perm_ref[j])),
        ],
        out_specs=pl.BlockSpec(
            (N_DEV * M_PER, N_PER), lambda j, k, perm_ref: (0, 0)
        ),
        scratch_shapes=[
            pltpu.VMEM((M_PER, N_PER), jnp.float32),
            pltpu.VMEM((N_DEV - 1, M_PER, N_PER), jnp.bfloat16),
            pltpu.SemaphoreType.DMA((N_DEV - 1,)),
            pltpu.SemaphoreType.DMA((N_DEV - 1,)),
        ],
    )

    out16 = pl.pallas_call(
        body,
        grid_spec=grid_spec,
        out_shape=jax.ShapeDtypeStruct((N_DEV * M_PER, N_PER), jnp.bfloat16),
        compiler_params=pltpu.CompilerParams(
            collective_id=0,
            vmem_limit_bytes=60 * 1024 * 1024,
        ),
    )(perm, x16, w16)
    return out16.astype(jnp.float32)


# device time: 232557 ns/iter; 1.2709x vs baseline; 1.2709x over previous
import jax
import jax.numpy as jnp
from jax import lax
from jax.experimental import pallas as pl
from jax.experimental.pallas import tpu as pltpu

N_DEV = 4
M_PER = 2048
K = 8192
N_PER = 1024
K_BLK = 1024
N_K = K // K_BLK

OFF = (2, 1, 3, 0)


def kernel(x, w_mat):
    x16 = x.astype(jnp.bfloat16)

    p = lax.axis_index("i").astype(jnp.int32)
    perm = (p + jnp.array(OFF, dtype=jnp.int32)) % N_DEV

    def body(perm_ref, x_ref, w_ref, out_ref, y_acc, send_buf,
             send_sems, recv_sems):
        j = pl.program_id(0)
        k = pl.program_id(1)
        me = lax.axis_index("i").astype(jnp.int32)

        @pl.when(jnp.logical_and(j == 0, k == 0))
        def _():
            barrier = pltpu.get_barrier_semaphore()
            for d in range(1, N_DEV):
                pl.semaphore_signal(
                    barrier, inc=1,
                    device_id=((me + d) % N_DEV,),
                    device_id_type=pl.DeviceIdType.MESH,
                )
            pl.semaphore_wait(barrier, N_DEV - 1)

        @pl.when(k == 0)
        def _():
            y_acc[...] = jnp.zeros_like(y_acc)
        y_acc[...] += jnp.dot(
            x_ref[...],
            w_ref[...].astype(jnp.bfloat16),
            preferred_element_type=jnp.float32,
        )

        @pl.when(k == N_K - 1)
        def _():
            c = perm_ref[j]

            @pl.when(c == me)
            def _():
                out_ref[pl.ds(me * M_PER, M_PER), :] = (
                    y_acc[...].astype(jnp.bfloat16)
                )

            @pl.when(c != me)
            def _():
                send_buf[j] = y_acc[...].astype(jnp.bfloat16)
                slot = (me - c - 1) % N_DEV
                rdma = pltpu.make_async_remote_copy(
                    src_ref=send_buf.at[j],
                    dst_ref=out_ref.at[pl.ds(me * M_PER, M_PER), :],
                    send_sem=send_sems.at[j],
                    recv_sem=recv_sems.at[slot],
                    device_id=(c,),
                    device_id_type=pl.DeviceIdType.MESH,
                )
                rdma.start()

        @pl.when(jnp.logical_and(j == N_DEV - 1, k == N_K - 1))
        def _():
            for d in (2, 3, 1):
                o = (me + d) % N_DEV
                slot = (o - me - 1) % N_DEV
                rdma = pltpu.make_async_remote_copy(
                    src_ref=send_buf.at[0],
                    dst_ref=out_ref.at[pl.ds(o * M_PER, M_PER), :],
                    send_sem=send_sems.at[0],
                    recv_sem=recv_sems.at[slot],
                    device_id=(me,),
                    device_id_type=pl.DeviceIdType.MESH,
                )
                rdma.wait_recv()
            for s in range(N_DEV - 1):
                rdma = pltpu.make_async_remote_copy(
                    src_ref=send_buf.at[s],
                    dst_ref=out_ref.at[pl.ds(me * M_PER, M_PER), :],
                    send_sem=send_sems.at[s],
                    recv_sem=recv_sems.at[0],
                    device_id=(me,),
                    device_id_type=pl.DeviceIdType.MESH,
                )
                rdma.wait_send()

    grid_spec = pltpu.PrefetchScalarGridSpec(
        num_scalar_prefetch=1,
        grid=(N_DEV, N_K),
        in_specs=[
            pl.BlockSpec((M_PER, K_BLK), lambda j, k, perm_ref: (0, k)),
            pl.BlockSpec((K_BLK, N_PER), lambda j, k, perm_ref: (k, perm_ref[j])),
        ],
        out_specs=pl.BlockSpec(
            (N_DEV * M_PER, N_PER), lambda j, k, perm_ref: (0, 0)
        ),
        scratch_shapes=[
            pltpu.VMEM((M_PER, N_PER), jnp.float32),
            pltpu.VMEM((N_DEV - 1, M_PER, N_PER), jnp.bfloat16),
            pltpu.SemaphoreType.DMA((N_DEV - 1,)),
            pltpu.SemaphoreType.DMA((N_DEV - 1,)),
        ],
    )

    out16 = pl.pallas_call(
        body,
        grid_spec=grid_spec,
        out_shape=jax.ShapeDtypeStruct((N_DEV * M_PER, N_PER), jnp.bfloat16),
        compiler_params=pltpu.CompilerParams(
            collective_id=0,
            vmem_limit_bytes=60 * 1024 * 1024,
        ),
    )(perm, x16, w_mat)
    return out16.astype(jnp.float32)


# device time: 211654 ns/iter; 1.3965x vs baseline; 1.0988x over previous
import jax
import jax.numpy as jnp
from jax import lax
from jax.experimental import pallas as pl
from jax.experimental.pallas import tpu as pltpu

N_DEV = 4
M_PER = 2048
K = 8192
N_PER = 1024
K_BLK = 512
N_K = K // K_BLK

OFF = (2, 1, 3, 0)


def kernel(x, w_mat):
    p = lax.axis_index("i").astype(jnp.int32)
    perm = (p + jnp.array(OFF, dtype=jnp.int32)) % N_DEV

    def body(perm_ref, x_ref, w_ref, out_ref, y_acc, send_buf,
             send_sems, recv_sems):
        j = pl.program_id(0)
        k = pl.program_id(1)
        me = lax.axis_index("i").astype(jnp.int32)

        @pl.when(jnp.logical_and(j == 0, k == 0))
        def _():
            barrier = pltpu.get_barrier_semaphore()
            for d in range(1, N_DEV):
                pl.semaphore_signal(
                    barrier, inc=1,
                    device_id=((me + d) % N_DEV,),
                    device_id_type=pl.DeviceIdType.MESH,
                )
            pl.semaphore_wait(barrier, N_DEV - 1)

        @pl.when(k == 0)
        def _():
            y_acc[...] = jnp.zeros_like(y_acc)
        y_acc[...] += jnp.dot(
            x_ref[...].astype(jnp.bfloat16),
            w_ref[...].astype(jnp.bfloat16),
            preferred_element_type=jnp.float32,
        )

        @pl.when(k == N_K - 1)
        def _():
            c = perm_ref[j]

            @pl.when(c == me)
            def _():
                out_ref[pl.ds(me * M_PER, M_PER), :] = (
                    y_acc[...].astype(jnp.bfloat16)
                )

            @pl.when(c != me)
            def _():
                send_buf[j] = y_acc[...].astype(jnp.bfloat16)
                slot = (me - c - 1) % N_DEV
                rdma = pltpu.make_async_remote_copy(
                    src_ref=send_buf.at[j],
                    dst_ref=out_ref.at[pl.ds(me * M_PER, M_PER), :],
                    send_sem=send_sems.at[j],
                    recv_sem=recv_sems.at[slot],
                    device_id=(c,),
                    device_id_type=pl.DeviceIdType.MESH,
                )
                rdma.start()

        @pl.when(jnp.logical_and(j == N_DEV - 1, k == N_K - 1))
        def _():
            for d in (2, 3, 1):
                o = (me + d) % N_DEV
                slot = (o - me - 1) % N_DEV
                rdma = pltpu.make_async_remote_copy(
                    src_ref=send_buf.at[0],
                    dst_ref=out_ref.at[pl.ds(o * M_PER, M_PER), :],
                    send_sem=send_sems.at[0],
                    recv_sem=recv_sems.at[slot],
                    device_id=(me,),
                    device_id_type=pl.DeviceIdType.MESH,
                )
                rdma.wait_recv()
            for s in range(N_DEV - 1):
                rdma = pltpu.make_async_remote_copy(
                    src_ref=send_buf.at[s],
                    dst_ref=out_ref.at[pl.ds(me * M_PER, M_PER), :],
                    send_sem=send_sems.at[s],
                    recv_sem=recv_sems.at[0],
                    device_id=(me,),
                    device_id_type=pl.DeviceIdType.MESH,
                )
                rdma.wait_send()

    grid_spec = pltpu.PrefetchScalarGridSpec(
        num_scalar_prefetch=1,
        grid=(N_DEV, N_K),
        in_specs=[
            pl.BlockSpec((M_PER, K_BLK), lambda j, k, perm_ref: (0, k)),
            pl.BlockSpec((K_BLK, N_PER), lambda j, k, perm_ref: (k, perm_ref[j])),
        ],
        out_specs=pl.BlockSpec(
            (N_DEV * M_PER, N_PER), lambda j, k, perm_ref: (0, 0)
        ),
        scratch_shapes=[
            pltpu.VMEM((M_PER, N_PER), jnp.float32),
            pltpu.VMEM((N_DEV - 1, M_PER, N_PER), jnp.bfloat16),
            pltpu.SemaphoreType.DMA((N_DEV - 1,)),
            pltpu.SemaphoreType.DMA((N_DEV - 1,)),
        ],
    )

    out16 = pl.pallas_call(
        body,
        grid_spec=grid_spec,
        out_shape=jax.ShapeDtypeStruct((N_DEV * M_PER, N_PER), jnp.bfloat16),
        compiler_params=pltpu.CompilerParams(
            collective_id=0,
            vmem_limit_bytes=60 * 1024 * 1024,
        ),
    )(perm, x, w_mat)
    return out16.astype(jnp.float32)


# device time: 196263 ns/iter; 1.5060x vs baseline; 1.0784x over previous
import jax
import jax.numpy as jnp
from jax import lax
from jax.experimental import pallas as pl
from jax.experimental.pallas import tpu as pltpu

N_DEV = 4
M_PER = 2048
K = 8192
N_PER = 1024
K_BLK = 1024
N_K = K // K_BLK

OFF = (2, 1, 3, 0)


def kernel(x, w_mat):
    p = lax.axis_index("i").astype(jnp.int32)
    perm = (p + jnp.array(OFF, dtype=jnp.int32)) % N_DEV

    def body(perm_ref, x_ref, w_ref, out_ref, y_acc, send_buf,
             send_sems, recv_sems):
        j = pl.program_id(0)
        k = pl.program_id(1)
        me = lax.axis_index("i").astype(jnp.int32)

        @pl.when(jnp.logical_and(j == 0, k == 0))
        def _():
            barrier = pltpu.get_barrier_semaphore()
            for d in range(1, N_DEV):
                pl.semaphore_signal(
                    barrier, inc=1,
                    device_id=((me + d) % N_DEV,),
                    device_id_type=pl.DeviceIdType.MESH,
                )
            pl.semaphore_wait(barrier, N_DEV - 1)

        @pl.when(k == 0)
        def _():
            y_acc[...] = jnp.zeros_like(y_acc)
        y_acc[...] += jnp.dot(
            x_ref[...].astype(jnp.bfloat16),
            w_ref[...].astype(jnp.bfloat16),
            preferred_element_type=jnp.float32,
        )

        @pl.when(k == N_K - 1)
        def _():
            c = perm_ref[j]

            @pl.when(c == me)
            def _():
                out_ref[pl.ds(me * M_PER, M_PER), :] = (
                    y_acc[...].astype(jnp.bfloat16)
                )

            @pl.when(c != me)
            def _():
                sslot = j % 2
                @pl.when(j >= 2)
                def _():
                    prev = pltpu.make_async_remote_copy(
                        src_ref=send_buf.at[sslot],
                        dst_ref=out_ref.at[pl.ds(me * M_PER, M_PER), :],
                        send_sem=send_sems.at[sslot],
                        recv_sem=recv_sems.at[0],
                        device_id=(c,),
                        device_id_type=pl.DeviceIdType.MESH,
                    )
                    prev.wait_send()
                send_buf[sslot] = y_acc[...].astype(jnp.bfloat16)
                slot = (me - c - 1) % N_DEV
                rdma = pltpu.make_async_remote_copy(
                    src_ref=send_buf.at[sslot],
                    dst_ref=out_ref.at[pl.ds(me * M_PER, M_PER), :],
                    send_sem=send_sems.at[sslot],
                    recv_sem=recv_sems.at[slot],
                    device_id=(c,),
                    device_id_type=pl.DeviceIdType.MESH,
                )
                rdma.start()

        @pl.when(jnp.logical_and(j == N_DEV - 1, k == N_K - 1))
        def _():
            for d in (2, 3, 1):
                o = (me + d) % N_DEV
                slot = (o - me - 1) % N_DEV
                rdma = pltpu.make_async_remote_copy(
                    src_ref=send_buf.at[0],
                    dst_ref=out_ref.at[pl.ds(o * M_PER, M_PER), :],
                    send_sem=send_sems.at[0],
                    recv_sem=recv_sems.at[slot],
                    device_id=(me,),
                    device_id_type=pl.DeviceIdType.MESH,
                )
                rdma.wait_recv()
            for s in range(2):
                rdma = pltpu.make_async_remote_copy(
                    src_ref=send_buf.at[s],
                    dst_ref=out_ref.at[pl.ds(me * M_PER, M_PER), :],
                    send_sem=send_sems.at[s],
                    recv_sem=recv_sems.at[0],
                    device_id=(me,),
                    device_id_type=pl.DeviceIdType.MESH,
                )
                rdma.wait_send()

    grid_spec = pltpu.PrefetchScalarGridSpec(
        num_scalar_prefetch=1,
        grid=(N_DEV, N_K),
        in_specs=[
            pl.BlockSpec((M_PER, K_BLK), lambda j, k, perm_ref: (0, k)),
            pl.BlockSpec((K_BLK, N_PER), lambda j, k, perm_ref: (k, perm_ref[j])),
        ],
        out_specs=pl.BlockSpec(
            (N_DEV * M_PER, N_PER), lambda j, k, perm_ref: (0, 0)
        ),
        scratch_shapes=[
            pltpu.VMEM((M_PER, N_PER), jnp.float32),
            pltpu.VMEM((2, M_PER, N_PER), jnp.bfloat16),
            pltpu.SemaphoreType.DMA((2,)),
            pltpu.SemaphoreType.DMA((N_DEV - 1,)),
        ],
    )

    out16 = pl.pallas_call(
        body,
        grid_spec=grid_spec,
        out_shape=jax.ShapeDtypeStruct((N_DEV * M_PER, N_PER), jnp.bfloat16),
        compiler_params=pltpu.CompilerParams(
            collective_id=0,
            vmem_limit_bytes=66584576,
        ),
    )(perm, x, w_mat)
    return out16
